# Initial kernel scaffold; baseline (speedup 1.0000x reference)
import jax
import jax.numpy as jnp
from jax.experimental import pallas as pl


def kernel(src_embedding, tgt_embedding, src, tgt, temperature, is_corr):
    raise NotImplementedError("write your pallas kernel here")


# speedup vs baseline: 1.7461x; 1.7461x over previous
import math

import jax
import jax.numpy as jnp
from jax.experimental import pallas as pl
from jax.experimental.pallas import tpu as pltpu

_BM = 512


def _fused_body(temp_ref, srcT_ref, tgt_emb_ref, src_ext_ref, tgt_ext_ref,
                corres_ref, weight_ref, acc_ref):
    ni = pl.program_id(1)
    n_cols = tgt_emb_ref.shape[2]
    dot = jax.lax.dot_general(
        srcT_ref[0].astype(jnp.bfloat16), tgt_emb_ref[0].astype(jnp.bfloat16),
        (((1,), (0,)), ((), ())),
        preferred_element_type=jnp.float32)
    inv_sqrt_d = jnp.float32(1.0 / math.sqrt(srcT_ref.shape[2]))
    z = (dot * inv_sqrt_d) * temp_ref[pl.program_id(0), 0]
    zmax = jnp.max(z, axis=1, keepdims=True)
    ssum = jnp.sum(jnp.exp(z - zmax), axis=1, keepdims=True)
    w = 1.0 / ssum
    col = jax.lax.broadcasted_iota(jnp.int32, z.shape, 1)
    idx = jnp.min(jnp.where(z == zmax, col, n_cols), axis=1, keepdims=True)
    corres_ref[0] = idx
    weight_ref[0] = w
    ohw = jnp.where(col == idx, w, 0.0)
    ygw = jax.lax.dot_general(
        tgt_ext_ref[0], ohw, (((1,), (1,)), ((), ())),
        preferred_element_type=jnp.float32)
    acc_blk = jax.lax.dot_general(
        ygw, src_ext_ref[0], (((1,), (1,)), ((), ())),
        preferred_element_type=jnp.float32)

    @pl.when(ni == 0)
    def _init():
        acc_ref[0] = acc_blk

    @pl.when(ni > 0)
    def _accum():
        acc_ref[0] += acc_blk


def _procrustes_finish(acc):
    eps = jnp.float32(1e-7)
    syx = acc[:, :3, :3]
    sy = acc[:, :3, 3]
    sx = acc[:, 3, :3]
    tw = acc[:, 3, 3]
    denom = tw + eps
    mu_y = sy / denom[:, None]
    mu_x = sx / denom[:, None]
    s_w = tw / denom
    cov = (syx / denom[:, None, None]
           + (s_w - 2.0)[:, None, None] * mu_y[:, :, None] * mu_x[:, None, :])
    U, _, Vt = jnp.linalg.svd(cov)
    sign = jnp.where(jnp.linalg.det(U) * jnp.linalg.det(jnp.swapaxes(Vt, -1, -2)) < 0,
                     -1.0, 1.0).astype(jnp.float32)
    d = jnp.stack([jnp.ones_like(sign), jnp.ones_like(sign), sign], axis=-1)
    R = jnp.einsum('bij,bjk->bik', U * d[:, None, :], Vt)
    rmux = jnp.einsum('bij,bj->bi', R, mu_x)
    T = mu_y[:, None, :] - rmux[:, :, None]
    return R.astype(jnp.float32), T.astype(jnp.float32)


def kernel(src_embedding, tgt_embedding, src, tgt, temperature, is_corr):
    B, D, N = src_embedding.shape
    bm = _BM
    nb = N // bm
    srcT = jnp.transpose(src_embedding, (0, 2, 1))
    ones = jnp.ones((B, 1, N), jnp.float32)
    zeros = jnp.zeros((B, 4, N), jnp.float32)
    src_ext = jnp.concatenate([src, ones, zeros], axis=1)
    tgt_ext = jnp.concatenate([tgt, ones, zeros], axis=1)

    corres2, weight2, acc = pl.pallas_call(
        _fused_body,
        grid=(B, nb),
        in_specs=[
            pl.BlockSpec((B, 1), lambda b, n: (0, 0),
                         memory_space=pltpu.SMEM),
            pl.BlockSpec((1, bm, D), lambda b, n: (b, n, 0)),
            pl.BlockSpec((1, D, N), lambda b, n: (b, 0, 0)),
            pl.BlockSpec((1, 8, bm), lambda b, n: (b, 0, n)),
            pl.BlockSpec((1, 8, N), lambda b, n: (b, 0, 0)),
        ],
        out_specs=[
            pl.BlockSpec((1, bm, 1), lambda b, n: (b, n, 0)),
            pl.BlockSpec((1, bm, 1), lambda b, n: (b, n, 0)),
            pl.BlockSpec((1, 8, 8), lambda b, n: (b, 0, 0)),
        ],
        out_shape=[
            jax.ShapeDtypeStruct((B, N, 1), jnp.int32),
            jax.ShapeDtypeStruct((B, N, 1), jnp.float32),
            jax.ShapeDtypeStruct((B, 8, 8), jnp.float32),
        ],
        compiler_params=pltpu.CompilerParams(
            dimension_semantics=("parallel", "arbitrary")),
    )(temperature.astype(jnp.float32).reshape(B, 1), srcT, tgt_embedding,
      src_ext, tgt_ext)

    R, T = _procrustes_finish(acc)
    return (R, T, corres2, weight2)
